# baseline (speedup 1.0000x reference)
import jax
import jax.numpy as jnp
from jax import lax
from jax.experimental import pallas as pl
from jax.experimental.pallas import tpu as pltpu
from jax.experimental.pallas import tpu_sc as plsc

N = 10000
D = 128
E = 320000
H = 32

NC = 2
NS = 16
NW = NC * NS

NP = 10240
CH = 128
CROWS = E // CH
RB = CROWS // NW
REXTRA = CROWS - RB * NW
RMAX = RB + 1
NBUF = 4
NRND = RB // NBUF
ROWS_PER_TILE = NP // NS


def _sc_mesh():
    return plsc.VectorSubcoreMesh(core_axis_name="c", subcore_axis_name="s")


CR = 6
CE = CR * CH
NCHK = RB // CR
PW = RB * CH


def _worker_rows(wid):
    base_row = wid * RB + jnp.minimum(wid, REXTRA)
    nrows = jnp.where(wid < REXTRA, RB + 1, RB)
    return base_row, nrows


def _preload_flat(e_flat, plane, base_e, dst_v):
    pltpu.sync_copy(e_flat.at[plane, pl.ds(base_e, PW)], dst_v.at[pl.ds(0, PW)])


def _preload_chunks(e_flat, plane, base_e, dst_v, wid, sem):
    for c in range(NCHK):
        pltpu.async_copy(e_flat.at[plane, pl.ds(base_e + c * CE, CE)],
                         dst_v.at[c], sem)
    for c in range(NCHK):
        pltpu.make_async_copy(e_flat.at[plane, pl.ds(base_e + c * CE, CE)],
                              dst_v.at[c], sem).wait()


def _preload_extra_flat(e_flat, plane, base_e, dst_v, off, wid):
    @pl.when(wid < REXTRA)
    def _():
        pltpu.sync_copy(e_flat.at[plane, pl.ds(base_e + PW, CH)],
                        dst_v.at[pl.ds(off, CH)])


def _deg_body(e_flat, degb_hbm, didx, ones_v, zv, deg_v, degb_v, deg_sh, sem):
    cid = lax.axis_index("c")
    sid = lax.axis_index("s")
    wid = cid * NS + sid
    base_row, nrows = _worker_rows(wid)
    base_e = base_row * CH

    _preload_chunks(e_flat, 1, base_e, didx, wid, sem)

    @pl.when(wid < REXTRA)
    def _():
        pltpu.sync_copy(e_flat.at[1, pl.ds(base_e + PW, CH)],
                        didx.at[NCHK, pl.ds(0, CH)])

    for k in range(CE // 16):
        ones_v[pl.ds(16 * k, 16)] = jnp.full((16,), 1.0, jnp.float32)
    for k in range(CH // 16):
        zv[pl.ds(16 * k, 16)] = jnp.zeros((16,), jnp.float32)
    for i in range(ROWS_PER_TILE // CH):
        pltpu.sync_copy(zv, deg_sh.at[pl.ds(sid * ROWS_PER_TILE + i * CH, CH)])
    plsc.subcore_barrier()

    for c in range(NCHK):
        pltpu.async_copy(ones_v, deg_sh.at[didx.at[c]], sem, add=True)
    for c in range(NCHK):
        pltpu.make_async_copy(ones_v, deg_sh.at[didx.at[c]], sem).wait()

    @pl.when(nrows == RMAX)
    def _():
        pltpu.sync_copy(ones_v.at[pl.ds(0, CH)],
                        deg_sh.at[didx.at[NCHK, pl.ds(0, CH)]], add=True)

    plsc.subcore_barrier()
    pltpu.sync_copy(
        deg_sh.at[pl.ds(sid * ROWS_PER_TILE, ROWS_PER_TILE)], deg_v
    )

    def brow(r, carry):
        row = plsc.load_gather(deg_v, [jnp.full((16,), r, jnp.int32)])
        for k in range(H // 16):
            degb_v[r, pl.ds(16 * k, 16)] = row
        return carry

    lax.fori_loop(0, ROWS_PER_TILE, brow, 0)
    pltpu.sync_copy(
        degb_v, degb_hbm.at[cid, pl.ds(sid * ROWS_PER_TILE, ROWS_PER_TILE)]
    )


@jax.jit
def _sc_deg(e_flat):
    return pl.kernel(
        _deg_body,
        out_type=jax.ShapeDtypeStruct((NC, NP, H), jnp.float32),
        mesh=_sc_mesh(),
        compiler_params=pltpu.CompilerParams(
            use_tc_tiling_on_sc=False, needs_layout_passes=False),
        scratch_types=[
            pltpu.VMEM((NCHK + 1, CE), jnp.int32),
            pltpu.VMEM((CE,), jnp.float32),
            pltpu.VMEM((CH,), jnp.float32),
            pltpu.VMEM((ROWS_PER_TILE,), jnp.float32),
            pltpu.VMEM((ROWS_PER_TILE, H), jnp.float32),
            pltpu.VMEM_SHARED((NP,), jnp.float32),
            pltpu.SemaphoreType.DMA,
        ],
    )(e_flat)


def _msg_body(e_flat, h_hbm, acc_hbm, sidx, didx, rows_v, acc_sh,
              sem_p, sem_g, sem_s):
    cid = lax.axis_index("c")
    sid = lax.axis_index("s")
    wid = cid * NS + sid
    base_row, nrows = _worker_rows(wid)
    base_e = base_row * CH

    _preload_flat(e_flat, 0, base_e, sidx)
    _preload_chunks(e_flat, 1, base_e, didx, wid, sem_p)
    _preload_extra_flat(e_flat, 0, base_e, sidx, PW, wid)

    @pl.when(wid < REXTRA)
    def _():
        pltpu.sync_copy(e_flat.at[1, pl.ds(base_e + PW, CH)],
                        didx.at[NCHK, pl.ds(0, CH)])

    def zrow(i, carry):
        rows_v[0, i, pl.ds(0, 16)] = jnp.zeros((16,), jnp.float32)
        rows_v[0, i, pl.ds(16, 16)] = jnp.zeros((16,), jnp.float32)
        return carry

    lax.fori_loop(0, CH, zrow, 0)
    for i in range(ROWS_PER_TILE // CH):
        pltpu.sync_copy(
            rows_v.at[0, pl.ds(0, CH)],
            acc_sh.at[pl.ds(sid * ROWS_PER_TILE + i * CH, CH)],
        )
    plsc.subcore_barrier()

    def g_start(c, b):
        pltpu.async_copy(h_hbm.at[sidx.at[pl.ds(c * CE, CE)]],
                         rows_v.at[b], sem_g.at[b])

    def g_wait(c, b):
        pltpu.make_async_copy(h_hbm.at[sidx.at[pl.ds(c * CE, CE)]],
                              rows_v.at[b], sem_g.at[b]).wait()

    def s_start(c, b):
        pltpu.async_copy(rows_v.at[b], acc_sh.at[didx.at[c]],
                         sem_s.at[b], add=True)

    def s_wait(c, b):
        pltpu.make_async_copy(rows_v.at[b], acc_sh.at[didx.at[c]],
                              sem_s.at[b]).wait()

    NB = 3
    G = 2
    for b in range(G):
        g_start(b, b)
    for c in range(NCHK):
        b = c % NB
        g_wait(c, b)
        s_start(c, b)
        if c + G < NCHK:
            if c - (NB - G) >= 0:
                s_wait(c - (NB - G), (c - (NB - G)) % NB)
            g_start(c + G, (c + G) % NB)

    for c in range(NCHK - 3, NCHK):
        s_wait(c, c % 3)

    @pl.when(nrows == RMAX)
    def _():
        pltpu.sync_copy(h_hbm.at[sidx.at[pl.ds(PW, CH)]],
                        rows_v.at[0, pl.ds(0, CH)])
        pltpu.sync_copy(rows_v.at[0, pl.ds(0, CH)],
                        acc_sh.at[didx.at[NCHK, pl.ds(0, CH)]], add=True)

    plsc.subcore_barrier()
    pltpu.sync_copy(
        acc_sh.at[pl.ds(sid * ROWS_PER_TILE, ROWS_PER_TILE)],
        acc_hbm.at[cid, pl.ds(sid * ROWS_PER_TILE, ROWS_PER_TILE)],
    )


@jax.jit
def _sc_msg(e_flat, h):
    return pl.kernel(
        _msg_body,
        out_type=jax.ShapeDtypeStruct((NC, NP, H), jnp.float32),
        mesh=_sc_mesh(),
        compiler_params=pltpu.CompilerParams(use_tc_tiling_on_sc=False),
        scratch_types=[
            pltpu.VMEM((PW + CH,), jnp.int32),
            pltpu.VMEM((NCHK + 1, CE), jnp.int32),
            pltpu.VMEM((3, CE, H), jnp.float32),
            pltpu.VMEM_SHARED((NP, H), jnp.float32),
            pltpu.SemaphoreType.DMA,
            pltpu.SemaphoreType.DMA((3,)),
            pltpu.SemaphoreType.DMA((3,)),
        ],
    )(e_flat, h)


VR = NP // 4
VRN = N // 4


def _blockdiag(w):
    t1 = jnp.concatenate([w, w, w, w], axis=0)
    t2 = jnp.concatenate([t1, t1, t1, t1], axis=1)
    ri = lax.broadcasted_iota(jnp.int32, (4 * H, 4 * H), 0)
    ci = lax.broadcasted_iota(jnp.int32, (4 * H, 4 * H), 1)
    return jnp.where((ri // H) == (ci // H), t2, 0.0)


def _tile4(b):
    return jnp.concatenate([b, b, b, b], axis=0)


def _tca_body(degb_ref, xv_ref, w1_ref, dinvbv_ref, h1sv_ref):
    dinvbv = lax.rsqrt(degb_ref[0] + degb_ref[1] + 1.0)
    t1 = jnp.concatenate([w1_ref[...]] * 4, axis=0)
    t2 = jnp.concatenate([t1] * 4, axis=1)
    ri = lax.broadcasted_iota(jnp.int32, (4 * H, 4 * D), 0)
    ci = lax.broadcasted_iota(jnp.int32, (4 * H, 4 * D), 1)
    w1bd = jnp.where((ri // H) == (ci // D), t2, 0.0)
    g1v = lax.dot_general(
        xv_ref[...], w1bd,
        dimension_numbers=(((1,), (1,)), ((), ())),
        preferred_element_type=jnp.float32,
    )
    g1vf = jnp.concatenate(
        [g1v, jnp.zeros((VR - VRN, 4 * H), jnp.float32)], axis=0)
    dinvbv_ref[...] = dinvbv
    h1sv_ref[...] = g1vf * dinvbv


@jax.jit
def _tc_a(degb, xv, w1):
    return pl.pallas_call(
        _tca_body,
        out_shape=(
            jax.ShapeDtypeStruct((VR, 4 * H), jnp.float32),
            jax.ShapeDtypeStruct((VR, 4 * H), jnp.float32),
        ),
    )(degb, xv, w1)


def _tcb_body(acc_ref, h1s_ref, dinvb_ref, b1_ref, w2_ref, h2s_ref):
    dinvb = dinvb_ref[...]
    pre = (dinvb * (acc_ref[0] + acc_ref[1] + h1s_ref[...])
           + _tile4(b1_ref[...])[None, :])
    act = jnp.tanh(pre)
    g2 = lax.dot_general(
        act, _blockdiag(w2_ref[...]),
        dimension_numbers=(((1,), (1,)), ((), ())),
        preferred_element_type=jnp.float32,
    )
    h2s_ref[...] = g2 * dinvb


@jax.jit
def _tc_b(acc, h1s, dinvb, b1, w2):
    return pl.pallas_call(
        _tcb_body,
        out_shape=jax.ShapeDtypeStruct((VR, 4 * H), jnp.float32),
    )(acc, h1s, dinvb, b1, w2)


def _tcc_body(acc_ref, h2s_ref, dinvb_ref, b2_ref, lw1_ref, lb1_ref,
              lw2_ref, lb2_ref, out_ref):
    pre = (dinvb_ref[...] * (acc_ref[0] + acc_ref[1] + h2s_ref[...])
           + _tile4(b2_ref[...])[None, :])
    act = jnp.tanh(pre)
    g3 = lax.dot_general(
        act, _blockdiag(lw1_ref[...]),
        dimension_numbers=(((1,), (1,)), ((), ())),
        preferred_element_type=jnp.float32,
    )
    h3 = jnp.tanh(g3 + _tile4(lb1_ref[...])[None, :])
    vr = lax.broadcasted_iota(jnp.int32, (VR, 4 * H), 0)
    h3 = jnp.where(vr < VRN, h3, 0.0)
    pooled = jnp.sum(h3, axis=0, keepdims=True)
    lw2t = jnp.concatenate([lw2_ref[...]] * 4, axis=1)
    out_ref[...] = (
        jnp.sum(pooled * lw2t, axis=1, keepdims=True) + lb2_ref[...][None, :]
    )


@jax.jit
def _tc_c(acc, h2s, dinvb, b2, lw1, lb1, lw2, lb2):
    return pl.pallas_call(
        _tcc_body,
        out_shape=jax.ShapeDtypeStruct((1, 1), jnp.float32),
    )(acc, h2s, dinvb, b2, lw1, lb1, lw2, lb2)


def kernel(x, edge_index, W1, b1, W2, b2, LW1, Lb1, LW2, Lb2):
    e_flat = edge_index.astype(jnp.int32)

    degb = _sc_deg(e_flat)
    xv = x.reshape(VRN, 4 * D)
    dinvbv, h1sv = _tc_a(degb.reshape(NC, VR, 4 * H), xv, W1)
    acc1 = _sc_msg(e_flat, h1sv.reshape(NP, H))
    h2sv = _tc_b(acc1.reshape(NC, VR, 4 * H), h1sv, dinvbv, b1, W2)
    acc2 = _sc_msg(e_flat, h2sv.reshape(NP, H))
    out = _tc_c(acc2.reshape(NC, VR, 4 * H), h2sv, dinvbv, b2, LW1, Lb1,
                LW2, Lb2)
    return out.reshape(1)


# speedup vs baseline: 1.0303x; 1.0303x over previous
import jax
import jax.numpy as jnp
from jax import lax
from jax.experimental import pallas as pl
from jax.experimental.pallas import tpu as pltpu
from jax.experimental.pallas import tpu_sc as plsc

N = 10000
D = 128
E = 320000
H = 32

NC = 2
NS = 16
NW = NC * NS

NP = 10240
CH = 128
CROWS = E // CH
RB = CROWS // NW
REXTRA = CROWS - RB * NW
RMAX = RB + 1
NBUF = 4
NRND = RB // NBUF
ROWS_PER_TILE = NP // NS


def _sc_mesh():
    return plsc.VectorSubcoreMesh(core_axis_name="c", subcore_axis_name="s")


CR = 6
CE = CR * CH
NCHK = RB // CR
PW = RB * CH


def _worker_rows(wid):
    base_row = wid * RB + jnp.minimum(wid, REXTRA)
    nrows = jnp.where(wid < REXTRA, RB + 1, RB)
    return base_row, nrows


def _preload_flat(e_flat, plane, base_e, dst_v):
    pltpu.sync_copy(e_flat.at[plane, pl.ds(base_e, PW)], dst_v.at[pl.ds(0, PW)])


def _preload_chunks(e_flat, plane, base_e, dst_v, wid, sem):
    for c in range(NCHK):
        pltpu.async_copy(e_flat.at[plane, pl.ds(base_e + c * CE, CE)],
                         dst_v.at[c], sem)
    for c in range(NCHK):
        pltpu.make_async_copy(e_flat.at[plane, pl.ds(base_e + c * CE, CE)],
                              dst_v.at[c], sem).wait()


def _preload_extra_flat(e_flat, plane, base_e, dst_v, off, wid):
    @pl.when(wid < REXTRA)
    def _():
        pltpu.sync_copy(e_flat.at[plane, pl.ds(base_e + PW, CH)],
                        dst_v.at[pl.ds(off, CH)])


def _deg_body(e_flat, degb_hbm, didx, ones_v, zv, deg_v, degb_v, deg_sh, sem):
    cid = lax.axis_index("c")
    sid = lax.axis_index("s")
    wid = cid * NS + sid
    base_row, nrows = _worker_rows(wid)
    base_e = base_row * CH

    _preload_chunks(e_flat, 1, base_e, didx, wid, sem)

    @pl.when(wid < REXTRA)
    def _():
        pltpu.sync_copy(e_flat.at[1, pl.ds(base_e + PW, CH)],
                        didx.at[NCHK, pl.ds(0, CH)])

    for k in range(CE // 16):
        ones_v[pl.ds(16 * k, 16)] = jnp.full((16,), 1.0, jnp.float32)
    for k in range(CH // 16):
        zv[pl.ds(16 * k, 16)] = jnp.zeros((16,), jnp.float32)
    for i in range(ROWS_PER_TILE // CH):
        pltpu.sync_copy(zv, deg_sh.at[pl.ds(sid * ROWS_PER_TILE + i * CH, CH)])
    plsc.subcore_barrier()

    for c in range(NCHK):
        pltpu.async_copy(ones_v, deg_sh.at[didx.at[c]], sem, add=True)
    for c in range(NCHK):
        pltpu.make_async_copy(ones_v, deg_sh.at[didx.at[c]], sem).wait()

    @pl.when(nrows == RMAX)
    def _():
        pltpu.sync_copy(ones_v.at[pl.ds(0, CH)],
                        deg_sh.at[didx.at[NCHK, pl.ds(0, CH)]], add=True)

    plsc.subcore_barrier()
    pltpu.sync_copy(
        deg_sh.at[pl.ds(sid * ROWS_PER_TILE, ROWS_PER_TILE)], deg_v
    )

    def brow(r, carry):
        row = plsc.load_gather(deg_v, [jnp.full((16,), r, jnp.int32)])
        for k in range(H // 16):
            degb_v[r, pl.ds(16 * k, 16)] = row
        return carry

    lax.fori_loop(0, ROWS_PER_TILE, brow, 0)
    pltpu.sync_copy(
        degb_v, degb_hbm.at[cid, pl.ds(sid * ROWS_PER_TILE, ROWS_PER_TILE)]
    )


@jax.jit
def _sc_deg(e_flat):
    return pl.kernel(
        _deg_body,
        out_type=jax.ShapeDtypeStruct((NC, NP, H), jnp.float32),
        mesh=_sc_mesh(),
        compiler_params=pltpu.CompilerParams(
            use_tc_tiling_on_sc=False, needs_layout_passes=False),
        scratch_types=[
            pltpu.VMEM((NCHK + 1, CE), jnp.int32),
            pltpu.VMEM((CE,), jnp.float32),
            pltpu.VMEM((CH,), jnp.float32),
            pltpu.VMEM((ROWS_PER_TILE,), jnp.float32),
            pltpu.VMEM((ROWS_PER_TILE, H), jnp.float32),
            pltpu.VMEM_SHARED((NP,), jnp.float32),
            pltpu.SemaphoreType.DMA,
        ],
    )(e_flat)


def _msg_body(e_flat, h_hbm, acc_hbm, sidx, didx, rows_v, acc_sh,
              sem_p, sem_g, sem_s):
    cid = lax.axis_index("c")
    sid = lax.axis_index("s")
    wid = cid * NS + sid
    base_row, nrows = _worker_rows(wid)
    base_e = base_row * CH

    _preload_flat(e_flat, 0, base_e, sidx)
    _preload_chunks(e_flat, 1, base_e, didx, wid, sem_p)
    _preload_extra_flat(e_flat, 0, base_e, sidx, PW, wid)

    @pl.when(wid < REXTRA)
    def _():
        pltpu.sync_copy(e_flat.at[1, pl.ds(base_e + PW, CH)],
                        didx.at[NCHK, pl.ds(0, CH)])

    def zrow(i, carry):
        rows_v[0, i, pl.ds(0, 16)] = jnp.zeros((16,), jnp.float32)
        rows_v[0, i, pl.ds(16, 16)] = jnp.zeros((16,), jnp.float32)
        return carry

    lax.fori_loop(0, CH, zrow, 0)
    for i in range(ROWS_PER_TILE // CH):
        pltpu.sync_copy(
            rows_v.at[0, pl.ds(0, CH)],
            acc_sh.at[pl.ds(sid * ROWS_PER_TILE + i * CH, CH)],
        )
    plsc.subcore_barrier()

    def g_start(c, b):
        pltpu.async_copy(h_hbm.at[sidx.at[pl.ds(c * CE, CE)]],
                         rows_v.at[b], sem_g.at[b])

    def g_wait(c, b):
        pltpu.make_async_copy(h_hbm.at[sidx.at[pl.ds(c * CE, CE)]],
                              rows_v.at[b], sem_g.at[b]).wait()

    def s_start(c, b):
        pltpu.async_copy(rows_v.at[b], acc_sh.at[didx.at[c]],
                         sem_s.at[b], add=True)

    def s_wait(c, b):
        pltpu.make_async_copy(rows_v.at[b], acc_sh.at[didx.at[c]],
                              sem_s.at[b]).wait()

    g_start(0, 0)
    for c in range(NCHK):
        b = c % 2
        bn = (c + 1) % 2
        if c >= 1:
            s_wait(c - 1, bn)
        if c + 1 < NCHK:
            g_start(c + 1, bn)
        g_wait(c, b)
        s_start(c, b)
    s_wait(NCHK - 1, (NCHK - 1) % 2)

    @pl.when(nrows == RMAX)
    def _():
        pltpu.sync_copy(h_hbm.at[sidx.at[pl.ds(PW, CH)]],
                        rows_v.at[0, pl.ds(0, CH)])
        pltpu.sync_copy(rows_v.at[0, pl.ds(0, CH)],
                        acc_sh.at[didx.at[NCHK, pl.ds(0, CH)]], add=True)

    plsc.subcore_barrier()
    pltpu.sync_copy(
        acc_sh.at[pl.ds(sid * ROWS_PER_TILE, ROWS_PER_TILE)],
        acc_hbm.at[cid, pl.ds(sid * ROWS_PER_TILE, ROWS_PER_TILE)],
    )


@jax.jit
def _sc_msg(e_flat, h):
    return pl.kernel(
        _msg_body,
        out_type=jax.ShapeDtypeStruct((NC, NP, H), jnp.float32),
        mesh=_sc_mesh(),
        compiler_params=pltpu.CompilerParams(use_tc_tiling_on_sc=False),
        scratch_types=[
            pltpu.VMEM((PW + CH,), jnp.int32),
            pltpu.VMEM((NCHK + 1, CE), jnp.int32),
            pltpu.VMEM((2, CE, H), jnp.float32),
            pltpu.VMEM_SHARED((NP, H), jnp.float32),
            pltpu.SemaphoreType.DMA,
            pltpu.SemaphoreType.DMA((2,)),
            pltpu.SemaphoreType.DMA((2,)),
        ],
    )(e_flat, h)


VR = NP // 4
VRN = N // 4


def _blockdiag(w):
    t1 = jnp.concatenate([w, w, w, w], axis=0)
    t2 = jnp.concatenate([t1, t1, t1, t1], axis=1)
    ri = lax.broadcasted_iota(jnp.int32, (4 * H, 4 * H), 0)
    ci = lax.broadcasted_iota(jnp.int32, (4 * H, 4 * H), 1)
    return jnp.where((ri // H) == (ci // H), t2, 0.0)


def _tile4(b):
    return jnp.concatenate([b, b, b, b], axis=0)


def _tca_body(degb_ref, xv_ref, w1_ref, dinvbv_ref, h1sv_ref):
    dinvbv = lax.rsqrt(degb_ref[0] + degb_ref[1] + 1.0)
    t1 = jnp.concatenate([w1_ref[...]] * 4, axis=0)
    t2 = jnp.concatenate([t1] * 4, axis=1)
    ri = lax.broadcasted_iota(jnp.int32, (4 * H, 4 * D), 0)
    ci = lax.broadcasted_iota(jnp.int32, (4 * H, 4 * D), 1)
    w1bd = jnp.where((ri // H) == (ci // D), t2, 0.0)
    g1v = lax.dot_general(
        xv_ref[...], w1bd,
        dimension_numbers=(((1,), (1,)), ((), ())),
        preferred_element_type=jnp.float32,
    )
    g1vf = jnp.concatenate(
        [g1v, jnp.zeros((VR - VRN, 4 * H), jnp.float32)], axis=0)
    dinvbv_ref[...] = dinvbv
    h1sv_ref[...] = g1vf * dinvbv


@jax.jit
def _tc_a(degb, xv, w1):
    return pl.pallas_call(
        _tca_body,
        out_shape=(
            jax.ShapeDtypeStruct((VR, 4 * H), jnp.float32),
            jax.ShapeDtypeStruct((VR, 4 * H), jnp.float32),
        ),
    )(degb, xv, w1)


def _tcb_body(acc_ref, h1s_ref, dinvb_ref, b1_ref, w2_ref, h2s_ref):
    dinvb = dinvb_ref[...]
    pre = (dinvb * (acc_ref[0] + acc_ref[1] + h1s_ref[...])
           + _tile4(b1_ref[...])[None, :])
    act = jnp.tanh(pre)
    g2 = lax.dot_general(
        act, _blockdiag(w2_ref[...]),
        dimension_numbers=(((1,), (1,)), ((), ())),
        preferred_element_type=jnp.float32,
    )
    h2s_ref[...] = g2 * dinvb


@jax.jit
def _tc_b(acc, h1s, dinvb, b1, w2):
    return pl.pallas_call(
        _tcb_body,
        out_shape=jax.ShapeDtypeStruct((VR, 4 * H), jnp.float32),
    )(acc, h1s, dinvb, b1, w2)


def _tcc_body(acc_ref, h2s_ref, dinvb_ref, b2_ref, lw1_ref, lb1_ref,
              lw2_ref, lb2_ref, out_ref):
    pre = (dinvb_ref[...] * (acc_ref[0] + acc_ref[1] + h2s_ref[...])
           + _tile4(b2_ref[...])[None, :])
    act = jnp.tanh(pre)
    g3 = lax.dot_general(
        act, _blockdiag(lw1_ref[...]),
        dimension_numbers=(((1,), (1,)), ((), ())),
        preferred_element_type=jnp.float32,
    )
    h3 = jnp.tanh(g3 + _tile4(lb1_ref[...])[None, :])
    vr = lax.broadcasted_iota(jnp.int32, (VR, 4 * H), 0)
    h3 = jnp.where(vr < VRN, h3, 0.0)
    pooled = jnp.sum(h3, axis=0, keepdims=True)
    lw2t = jnp.concatenate([lw2_ref[...]] * 4, axis=1)
    out_ref[...] = (
        jnp.sum(pooled * lw2t, axis=1, keepdims=True) + lb2_ref[...][None, :]
    )


@jax.jit
def _tc_c(acc, h2s, dinvb, b2, lw1, lb1, lw2, lb2):
    return pl.pallas_call(
        _tcc_body,
        out_shape=jax.ShapeDtypeStruct((1, 1), jnp.float32),
    )(acc, h2s, dinvb, b2, lw1, lb1, lw2, lb2)


def kernel(x, edge_index, W1, b1, W2, b2, LW1, Lb1, LW2, Lb2):
    e_flat = edge_index.astype(jnp.int32)

    degb = _sc_deg(e_flat)
    xv = x.reshape(VRN, 4 * D)
    dinvbv, h1sv = _tc_a(degb.reshape(NC, VR, 4 * H), xv, W1)
    acc1 = _sc_msg(e_flat, h1sv.reshape(NP, H))
    h2sv = _tc_b(acc1.reshape(NC, VR, 4 * H), h1sv, dinvbv, b1, W2)
    acc2 = _sc_msg(e_flat, h2sv.reshape(NP, H))
    out = _tc_c(acc2.reshape(NC, VR, 4 * H), h2sv, dinvbv, b2, LW1, Lb1,
                LW2, Lb2)
    return out.reshape(1)


# speedup vs baseline: 1.0326x; 1.0022x over previous
import jax
import jax.numpy as jnp
from jax import lax
from jax.experimental import pallas as pl
from jax.experimental.pallas import tpu as pltpu
from jax.experimental.pallas import tpu_sc as plsc

N = 10000
D = 128
E = 320000
H = 32

NC = 2
NS = 16
NW = NC * NS

NP = 10240
CH = 128
CROWS = E // CH
RB = CROWS // NW
REXTRA = CROWS - RB * NW
RMAX = RB + 1
ROWS_PER_TILE = NP // NS


def _sc_mesh():
    return plsc.VectorSubcoreMesh(core_axis_name="c", subcore_axis_name="s")


CR = 6
CE = CR * CH
NCHK = RB // CR
PW = RB * CH


def _worker_rows(wid):
    base_row = wid * RB + jnp.minimum(wid, REXTRA)
    nrows = jnp.where(wid < REXTRA, RB + 1, RB)
    return base_row, nrows


def _preload_flat(e_flat, plane, base_e, dst_v):
    pltpu.sync_copy(e_flat.at[plane, pl.ds(base_e, PW)], dst_v.at[pl.ds(0, PW)])


def _preload_chunks(e_flat, plane, base_e, dst_v, wid, sem):
    for c in range(NCHK):
        pltpu.async_copy(e_flat.at[plane, pl.ds(base_e + c * CE, CE)],
                         dst_v.at[c], sem)
    for c in range(NCHK):
        pltpu.make_async_copy(e_flat.at[plane, pl.ds(base_e + c * CE, CE)],
                              dst_v.at[c], sem).wait()


def _preload_extra_flat(e_flat, plane, base_e, dst_v, off, wid):
    @pl.when(wid < REXTRA)
    def _():
        pltpu.sync_copy(e_flat.at[plane, pl.ds(base_e + PW, CH)],
                        dst_v.at[pl.ds(off, CH)])


def _deg_body(e_flat, degb_hbm, didx, ones_v, zv, deg_v, degb_v, deg_sh, sem):
    cid = lax.axis_index("c")
    sid = lax.axis_index("s")
    wid = cid * NS + sid
    base_row, nrows = _worker_rows(wid)
    base_e = base_row * CH

    _preload_chunks(e_flat, 1, base_e, didx, wid, sem)

    @pl.when(wid < REXTRA)
    def _():
        pltpu.sync_copy(e_flat.at[1, pl.ds(base_e + PW, CH)],
                        didx.at[NCHK, pl.ds(0, CH)])

    for k in range(CE // 16):
        ones_v[pl.ds(16 * k, 16)] = jnp.full((16,), 1.0, jnp.float32)
    for k in range(CH // 16):
        zv[pl.ds(16 * k, 16)] = jnp.zeros((16,), jnp.float32)
    for i in range(ROWS_PER_TILE // CH):
        pltpu.sync_copy(zv, deg_sh.at[pl.ds(sid * ROWS_PER_TILE + i * CH, CH)])
    plsc.subcore_barrier()

    for c in range(NCHK):
        pltpu.async_copy(ones_v, deg_sh.at[didx.at[c]], sem, add=True)
    for c in range(NCHK):
        pltpu.make_async_copy(ones_v, deg_sh.at[didx.at[c]], sem).wait()

    @pl.when(nrows == RMAX)
    def _():
        pltpu.sync_copy(ones_v.at[pl.ds(0, CH)],
                        deg_sh.at[didx.at[NCHK, pl.ds(0, CH)]], add=True)

    plsc.subcore_barrier()
    pltpu.sync_copy(
        deg_sh.at[pl.ds(sid * ROWS_PER_TILE, ROWS_PER_TILE)], deg_v
    )

    def brow(r, carry):
        row = plsc.load_gather(deg_v, [jnp.full((16,), r, jnp.int32)])
        for k in range(H // 16):
            degb_v[r, pl.ds(16 * k, 16)] = row
        return carry

    lax.fori_loop(0, ROWS_PER_TILE, brow, 0)
    pltpu.sync_copy(
        degb_v, degb_hbm.at[cid, pl.ds(sid * ROWS_PER_TILE, ROWS_PER_TILE)]
    )


@jax.jit
def _sc_deg(e_flat):
    return pl.kernel(
        _deg_body,
        out_type=jax.ShapeDtypeStruct((NC, NP, H), jnp.float32),
        mesh=_sc_mesh(),
        compiler_params=pltpu.CompilerParams(
            use_tc_tiling_on_sc=False, needs_layout_passes=False),
        scratch_types=[
            pltpu.VMEM((NCHK + 1, CE), jnp.int32),
            pltpu.VMEM((CE,), jnp.float32),
            pltpu.VMEM((CH,), jnp.float32),
            pltpu.VMEM((ROWS_PER_TILE,), jnp.float32),
            pltpu.VMEM((ROWS_PER_TILE, H), jnp.float32),
            pltpu.VMEM_SHARED((NP,), jnp.float32),
            pltpu.SemaphoreType.DMA,
        ],
    )(e_flat)


def _msg_body(e_flat, h_hbm, acc_hbm, sidx, didx, rows_v, acc_sh,
              sem_p, sem_g, sem_s):
    cid = lax.axis_index("c")
    sid = lax.axis_index("s")
    wid = cid * NS + sid
    base_row, nrows = _worker_rows(wid)
    base_e = base_row * CH

    _preload_flat(e_flat, 0, base_e, sidx)
    _preload_chunks(e_flat, 1, base_e, didx, wid, sem_p)
    _preload_extra_flat(e_flat, 0, base_e, sidx, PW, wid)

    @pl.when(wid < REXTRA)
    def _():
        pltpu.sync_copy(e_flat.at[1, pl.ds(base_e + PW, CH)],
                        didx.at[NCHK, pl.ds(0, CH)])

    def zrow(i, carry):
        rows_v[0, i, pl.ds(0, 16)] = jnp.zeros((16,), jnp.float32)
        rows_v[0, i, pl.ds(16, 16)] = jnp.zeros((16,), jnp.float32)
        return carry

    lax.fori_loop(0, CH, zrow, 0)
    for i in range(ROWS_PER_TILE // CH):
        pltpu.sync_copy(
            rows_v.at[0, pl.ds(0, CH)],
            acc_sh.at[pl.ds(sid * ROWS_PER_TILE + i * CH, CH)],
        )
    plsc.subcore_barrier()

    def g_start(c, b):
        pltpu.async_copy(h_hbm.at[sidx.at[pl.ds(c * CE, CE)]],
                         rows_v.at[b], sem_g.at[b])

    def g_wait(c, b):
        pltpu.make_async_copy(h_hbm.at[sidx.at[pl.ds(c * CE, CE)]],
                              rows_v.at[b], sem_g.at[b]).wait()

    def s_start(c, b):
        pltpu.async_copy(rows_v.at[b], acc_sh.at[didx.at[c]],
                         sem_s.at[b], add=True)

    def s_wait(c, b):
        pltpu.make_async_copy(rows_v.at[b], acc_sh.at[didx.at[c]],
                              sem_s.at[b]).wait()

    g_start(0, 0)
    for c in range(NCHK):
        b = c % 2
        bn = (c + 1) % 2
        if c >= 1:
            s_wait(c - 1, bn)
        if c + 1 < NCHK:
            g_start(c + 1, bn)
        g_wait(c, b)
        s_start(c, b)
    s_wait(NCHK - 1, (NCHK - 1) % 2)

    @pl.when(nrows == RMAX)
    def _():
        pltpu.sync_copy(h_hbm.at[sidx.at[pl.ds(PW, CH)]],
                        rows_v.at[0, pl.ds(0, CH)])
        pltpu.sync_copy(rows_v.at[0, pl.ds(0, CH)],
                        acc_sh.at[didx.at[NCHK, pl.ds(0, CH)]], add=True)

    plsc.subcore_barrier()
    pltpu.sync_copy(
        acc_sh.at[pl.ds(sid * ROWS_PER_TILE, ROWS_PER_TILE)],
        acc_hbm.at[cid, pl.ds(sid * ROWS_PER_TILE, ROWS_PER_TILE)],
    )


@jax.jit
def _sc_msg(e_flat, h):
    return pl.kernel(
        _msg_body,
        out_type=jax.ShapeDtypeStruct((NC, NP, H), jnp.float32),
        mesh=_sc_mesh(),
        compiler_params=pltpu.CompilerParams(use_tc_tiling_on_sc=False),
        scratch_types=[
            pltpu.VMEM((PW + CH,), jnp.int32),
            pltpu.VMEM((NCHK + 1, CE), jnp.int32),
            pltpu.VMEM((2, CE, H), jnp.float32),
            pltpu.VMEM_SHARED((NP, H), jnp.float32),
            pltpu.SemaphoreType.DMA,
            pltpu.SemaphoreType.DMA((2,)),
            pltpu.SemaphoreType.DMA((2,)),
        ],
    )(e_flat, h)


VR = NP // 4
VRN = N // 4


def _blockdiag(w):
    t1 = jnp.concatenate([w, w, w, w], axis=0)
    t2 = jnp.concatenate([t1, t1, t1, t1], axis=1)
    ri = lax.broadcasted_iota(jnp.int32, (4 * H, 4 * H), 0)
    ci = lax.broadcasted_iota(jnp.int32, (4 * H, 4 * H), 1)
    return jnp.where((ri // H) == (ci // H), t2, 0.0)


def _tile4(b):
    return jnp.concatenate([b, b, b, b], axis=0)


def _tca_body(degb_ref, xv_ref, w1_ref, dinvbv_ref, h1sv_ref):
    dinvbv = lax.rsqrt(degb_ref[0] + degb_ref[1] + 1.0)
    t1 = jnp.concatenate([w1_ref[...]] * 4, axis=0)
    t2 = jnp.concatenate([t1] * 4, axis=1)
    ri = lax.broadcasted_iota(jnp.int32, (4 * H, 4 * D), 0)
    ci = lax.broadcasted_iota(jnp.int32, (4 * H, 4 * D), 1)
    w1bd = jnp.where((ri // H) == (ci // D), t2, 0.0)
    g1v = lax.dot_general(
        xv_ref[...], w1bd,
        dimension_numbers=(((1,), (1,)), ((), ())),
        preferred_element_type=jnp.float32,
    )
    g1vf = jnp.concatenate(
        [g1v, jnp.zeros((VR - VRN, 4 * H), jnp.float32)], axis=0)
    dinvbv_ref[...] = dinvbv
    h1sv_ref[...] = g1vf * dinvbv


@jax.jit
def _tc_a(degb, xv, w1):
    return pl.pallas_call(
        _tca_body,
        out_shape=(
            jax.ShapeDtypeStruct((VR, 4 * H), jnp.float32),
            jax.ShapeDtypeStruct((VR, 4 * H), jnp.float32),
        ),
    )(degb, xv, w1)


def _tcb_body(acc_ref, h1s_ref, dinvb_ref, b1_ref, w2_ref, h2s_ref):
    dinvb = dinvb_ref[...]
    pre = (dinvb * (acc_ref[0] + acc_ref[1] + h1s_ref[...])
           + _tile4(b1_ref[...])[None, :])
    act = jnp.tanh(pre)
    g2 = lax.dot_general(
        act, _blockdiag(w2_ref[...]),
        dimension_numbers=(((1,), (1,)), ((), ())),
        preferred_element_type=jnp.float32,
    )
    h2s_ref[...] = g2 * dinvb


@jax.jit
def _tc_b(acc, h1s, dinvb, b1, w2):
    return pl.pallas_call(
        _tcb_body,
        out_shape=jax.ShapeDtypeStruct((VR, 4 * H), jnp.float32),
    )(acc, h1s, dinvb, b1, w2)


def _tcc_body(acc_ref, h2s_ref, dinvb_ref, b2_ref, lw1_ref, lb1_ref,
              lw2_ref, lb2_ref, out_ref):
    pre = (dinvb_ref[...] * (acc_ref[0] + acc_ref[1] + h2s_ref[...])
           + _tile4(b2_ref[...])[None, :])
    act = jnp.tanh(pre)
    g3 = lax.dot_general(
        act, _blockdiag(lw1_ref[...]),
        dimension_numbers=(((1,), (1,)), ((), ())),
        preferred_element_type=jnp.float32,
    )
    h3 = jnp.tanh(g3 + _tile4(lb1_ref[...])[None, :])
    vr = lax.broadcasted_iota(jnp.int32, (VR, 4 * H), 0)
    h3 = jnp.where(vr < VRN, h3, 0.0)
    pooled = jnp.sum(h3, axis=0, keepdims=True)
    lw2t = jnp.concatenate([lw2_ref[...]] * 4, axis=1)
    out_ref[...] = (
        jnp.sum(pooled * lw2t, axis=1, keepdims=True) + lb2_ref[...][None, :]
    )


@jax.jit
def _tc_c(acc, h2s, dinvb, b2, lw1, lb1, lw2, lb2):
    return pl.pallas_call(
        _tcc_body,
        out_shape=jax.ShapeDtypeStruct((1, 1), jnp.float32),
    )(acc, h2s, dinvb, b2, lw1, lb1, lw2, lb2)


def kernel(x, edge_index, W1, b1, W2, b2, LW1, Lb1, LW2, Lb2):
    e_flat = edge_index.astype(jnp.int32)

    degb = _sc_deg(e_flat)
    xv = x.reshape(VRN, 4 * D)
    dinvbv, h1sv = _tc_a(degb.reshape(NC, VR, 4 * H), xv, W1)
    acc1 = _sc_msg(e_flat, h1sv.reshape(NP, H))
    h2sv = _tc_b(acc1.reshape(NC, VR, 4 * H), h1sv, dinvbv, b1, W2)
    acc2 = _sc_msg(e_flat, h2sv.reshape(NP, H))
    out = _tc_c(acc2.reshape(NC, VR, 4 * H), h2sv, dinvbv, b2, LW1, Lb1,
                LW2, Lb2)
    return out.reshape(1)
